# baseline (speedup 1.0000x reference)
import functools

import jax
import jax.numpy as jnp
from jax import lax
from jax.experimental import pallas as pl
from jax.experimental.pallas import tpu as pltpu
from jax.experimental.pallas import tpu_sc as plsc

N = 1_000_000
K = 16_384
L = 16
NC, NS = 2, 16
NW = NC * NS
SHARD = 31_256
LAST_BASE = N - SHARD
CHUNKS = K // L


_mesh = plsc.VectorSubcoreMesh(core_axis_name="c", subcore_axis_name="s")


@functools.partial(
    pl.kernel,
    mesh=_mesh,
    out_type=jax.ShapeDtypeStruct((N,), jnp.float32),
    scratch_types=[
        pltpu.VMEM((SHARD,), jnp.float32),
        pltpu.VMEM((K,), jnp.int32),
        pltpu.VMEM((K,), jnp.float32),
    ],
    compiler_params=pltpu.CompilerParams(needs_layout_passes=False),
)
def _scatter_set(in_hbm, idx_hbm, val_hbm, out_hbm, shard_v, idx_v, val_v):
    wid = lax.axis_index("s") * NC + lax.axis_index("c")
    base = jnp.where(wid == NW - 1, LAST_BASE, wid * SHARD)

    pltpu.sync_copy(idx_hbm, idx_v)
    pltpu.sync_copy(val_hbm, val_v)
    pltpu.sync_copy(in_hbm.at[pl.ds(base, SHARD)], shard_v)

    lane = lax.broadcasted_iota(jnp.int32, (L,), 0)

    def body(c, carry):
        s = c * L
        iv = idx_v[pl.ds(s, L)]
        vv = val_v[pl.ds(s, L)]
        m = (iv >= base) & (iv < base + SHARD)
        rel = jnp.where(m, iv - base, 0)
        plsc.store_scatter(shard_v, [rel], vv, mask=m)
        back = plsc.load_gather(shard_v, [rel], mask=m)
        bad = jnp.any((back != vv) & m)

        @pl.when(bad)
        def _fix():
            for i in range(L):
                plsc.store_scatter(shard_v, [rel], vv, mask=m & (lane == i))

        return carry

    lax.fori_loop(0, CHUNKS, body, 0)

    pltpu.sync_copy(shard_v, out_hbm.at[pl.ds(base, SHARD)])


def kernel(input, index, value):
    return _scatter_set(input, index.astype(jnp.int32), value)


# speedup vs baseline: 1.5236x; 1.5236x over previous
import functools

import jax
import jax.numpy as jnp
from jax import lax
from jax.experimental import pallas as pl
from jax.experimental.pallas import tpu as pltpu
from jax.experimental.pallas import tpu_sc as plsc

N = 1_000_000
K = 16_384
L = 16
NC, NS = 2, 16
NW = NC * NS
SHARD = 31_256
LAST_BASE = N - SHARD
CHUNKS = K // L


_mesh = plsc.VectorSubcoreMesh(core_axis_name="c", subcore_axis_name="s")


@functools.partial(
    pl.kernel,
    mesh=_mesh,
    out_type=jax.ShapeDtypeStruct((N,), jnp.float32),
    scratch_types=[
        pltpu.VMEM((SHARD,), jnp.float32),
        pltpu.VMEM((K,), jnp.int32),
        pltpu.VMEM((K,), jnp.float32),
    ],
    compiler_params=pltpu.CompilerParams(needs_layout_passes=False),
)
def _scatter_set(in_hbm, idx_hbm, val_hbm, out_hbm, shard_v, idx_v, val_v):
    wid = lax.axis_index("s") * NC + lax.axis_index("c")
    base = jnp.where(wid == NW - 1, LAST_BASE, wid * SHARD)

    pltpu.sync_copy(idx_hbm, idx_v)
    pltpu.sync_copy(val_hbm, val_v)
    pltpu.sync_copy(in_hbm.at[pl.ds(base, SHARD)], shard_v)

    def body(c, carry):
        s = c * L
        iv = idx_v[pl.ds(s, L)]
        vv = val_v[pl.ds(s, L)]
        m = (iv >= base) & (iv < base + SHARD)
        rel = jnp.where(m, iv - base, 0)
        _, last = plsc.scan_count(rel, m)
        plsc.store_scatter(shard_v, [rel], vv, mask=last & m)
        return carry

    lax.fori_loop(0, CHUNKS, body, 0, unroll=8)

    pltpu.sync_copy(shard_v, out_hbm.at[pl.ds(base, SHARD)])


def kernel(input, index, value):
    return _scatter_set(input, index.astype(jnp.int32), value)


# speedup vs baseline: 1.5440x; 1.0134x over previous
import functools

import jax
import jax.numpy as jnp
from jax import lax
from jax.experimental import pallas as pl
from jax.experimental.pallas import tpu as pltpu
from jax.experimental.pallas import tpu_sc as plsc

N = 1_000_000
K = 16_384
L = 16
NC, NS = 2, 16
NW = NC * NS
SHARD = 31_256
LAST_BASE = N - SHARD
CHUNKS = K // L


_mesh = plsc.VectorSubcoreMesh(core_axis_name="c", subcore_axis_name="s")


@functools.partial(
    pl.kernel,
    mesh=_mesh,
    out_type=jax.ShapeDtypeStruct((N,), jnp.float32),
    scratch_types=[
        pltpu.VMEM((SHARD,), jnp.float32),
        pltpu.VMEM((K,), jnp.int32),
        pltpu.VMEM((K,), jnp.float32),
    ],
    compiler_params=pltpu.CompilerParams(needs_layout_passes=False),
)
def _scatter_set(in_hbm, idx_hbm, val_hbm, out_hbm, shard_v, idx_v, val_v):
    wid = lax.axis_index("s") * NC + lax.axis_index("c")
    base = jnp.where(wid == NW - 1, LAST_BASE, wid * SHARD)

    pltpu.sync_copy(idx_hbm, idx_v)
    pltpu.sync_copy(val_hbm, val_v)
    pltpu.sync_copy(in_hbm.at[pl.ds(base, SHARD)], shard_v)

    def body(c, carry):
        s = c * L
        iv = idx_v[pl.ds(s, L)]
        vv = val_v[pl.ds(s, L)]
        rel = iv - base
        m = plsc.bitcast(rel, jnp.uint32) < jnp.uint32(SHARD)
        _, last = plsc.scan_count(rel, m)
        plsc.store_scatter(shard_v, [rel], vv, mask=last & m)
        return carry

    lax.fori_loop(0, CHUNKS, body, 0, unroll=16)

    pltpu.sync_copy(shard_v, out_hbm.at[pl.ds(base, SHARD)])


def kernel(input, index, value):
    return _scatter_set(input, index.astype(jnp.int32), value)


# speedup vs baseline: 1.7019x; 1.1023x over previous
import functools

import jax
import jax.numpy as jnp
from jax import lax
from jax.experimental import pallas as pl
from jax.experimental.pallas import tpu as pltpu
from jax.experimental.pallas import tpu_sc as plsc

N = 1_000_000
K = 16_384
L = 16
NC, NS = 2, 16
NW = NC * NS
SHARD = 31_256
LAST_BASE = N - SHARD
CHUNKS = K // L


_mesh = plsc.VectorSubcoreMesh(core_axis_name="c", subcore_axis_name="s")


@functools.partial(
    pl.kernel,
    mesh=_mesh,
    out_type=jax.ShapeDtypeStruct((N,), jnp.float32),
    scratch_types=[
        pltpu.VMEM((SHARD,), jnp.float32),
        pltpu.VMEM((K,), jnp.int32),
        pltpu.VMEM((K,), jnp.float32),
        pltpu.VMEM_SHARED((K,), jnp.int32),
        pltpu.VMEM_SHARED((K,), jnp.float32),
        pltpu.SemaphoreType.DMA,
    ],
    compiler_params=pltpu.CompilerParams(needs_layout_passes=False),
)
def _scatter_set(in_hbm, idx_hbm, val_hbm, out_hbm, shard_v, idx_v, val_v,
                 idx_sh, val_sh, sem):
    cid = lax.axis_index("c")
    sid = lax.axis_index("s")
    wid = sid * NC + cid
    base = jnp.where(wid == NW - 1, LAST_BASE, wid * SHARD)

    shard_cpy = pltpu.async_copy(in_hbm.at[pl.ds(base, SHARD)], shard_v, sem)

    kslice = K // NS
    off = sid * kslice
    pltpu.sync_copy(idx_hbm.at[pl.ds(off, kslice)], idx_sh.at[pl.ds(off, kslice)])
    pltpu.sync_copy(val_hbm.at[pl.ds(off, kslice)], val_sh.at[pl.ds(off, kslice)])
    plsc.subcore_barrier()
    pltpu.sync_copy(idx_sh, idx_v)
    pltpu.sync_copy(val_sh, val_v)
    shard_cpy.wait()

    def body(c, carry):
        s = c * L
        iv = idx_v[pl.ds(s, L)]
        vv = val_v[pl.ds(s, L)]
        rel = iv - base
        m = plsc.bitcast(rel, jnp.uint32) < jnp.uint32(SHARD)
        _, last = plsc.scan_count(rel, m)
        plsc.store_scatter(shard_v, [rel], vv, mask=last & m)
        return carry

    lax.fori_loop(0, CHUNKS, body, 0, unroll=16)

    pltpu.sync_copy(shard_v, out_hbm.at[pl.ds(base, SHARD)])


def kernel(input, index, value):
    return _scatter_set(input, index.astype(jnp.int32), value)
